# baseline (speedup 1.0000x reference)
import functools

import jax
import jax.numpy as jnp
from jax import lax
from jax.experimental import pallas as pl
from jax.experimental.pallas import tpu as pltpu
from jax.experimental.pallas import tpu_sc as plsc

N, E, D, PC, NH = 10000, 160000, 256, 256, 512
HALF = 128
NC, NS = 2, 16
K = 128
NBUF = 2
NPASS = 2
EPT = -(-E // (NS * K * NBUF * NPASS)) * K * NBUF * NPASS
EPAD = EPT * NS
NCHUNK = EPT // K
PCH = NCHUNK // NPASS
ROWS_PT = (N // NS) // 8 * 8
TAIL = N - ROWS_PT * NS
APAD = 8


def _sc_segment_sum(x2, xh, gidx, didx):
    mesh = plsc.VectorSubcoreMesh(core_axis_name="c", subcore_axis_name="s")

    @functools.partial(
        pl.kernel,
        mesh=mesh,
        out_type=jax.ShapeDtypeStruct((NC, N, HALF), jnp.float32),
        scratch_types=[
            pltpu.VMEM((PCH, K), jnp.int32),
            pltpu.VMEM((PCH, K), jnp.int32),
            pltpu.VMEM((NBUF, K, HALF), jnp.float32),
            pltpu.VMEM_SHARED((N + APAD, HALF), jnp.float32),
        ] + [pltpu.SemaphoreType.DMA] * (2 * NBUF),
    )
    def k(x2_hbm, xh_hbm, gidx_hbm, didx_hbm, out_hbm,
          gv, dv, rows, agg, *sems):
        gsem, ssem = sems[:NBUF], sems[NBUF:]
        c = lax.axis_index("c")
        s = lax.axis_index("s")
        r0 = s * ROWS_PT
        pltpu.sync_copy(xh_hbm.at[c, pl.ds(r0, ROWS_PT)],
                        agg.at[pl.ds(r0, ROWS_PT)])

        @pl.when(s == 0)
        def _():
            pltpu.sync_copy(xh_hbm.at[c, pl.ds(ROWS_PT * NS, TAIL)],
                            agg.at[pl.ds(ROWS_PT * NS, TAIL)])

        plsc.subcore_barrier()

        for p in range(NPASS):
            cb = s * NCHUNK + p * PCH
            pltpu.sync_copy(gidx_hbm.at[c, pl.ds(cb, PCH)], gv)
            pltpu.sync_copy(didx_hbm.at[pl.ds(cb, PCH)], dv)

            for b in range(NBUF):
                pltpu.async_copy(x2_hbm.at[gv.at[b]], rows.at[b], gsem[b])

            def outer(i, carry):
                g = i * NBUF
                for b in range(NBUF):
                    j = g + b
                    pltpu.make_async_copy(x2_hbm.at[gv.at[j]],
                                          rows.at[b], gsem[b]).wait()
                    pltpu.async_copy(rows.at[b], agg.at[dv.at[j]],
                                     ssem[b], add=True)
                for b in range(NBUF):
                    j = g + b
                    pltpu.make_async_copy(rows.at[b], agg.at[dv.at[j]],
                                          ssem[b]).wait()
                    pltpu.async_copy(x2_hbm.at[gv.at[j + NBUF]],
                                     rows.at[b], gsem[b])
                return carry

            lax.fori_loop(0, PCH // NBUF - 1, outer, 0)
            gt = PCH - NBUF
            for b in range(NBUF):
                pltpu.make_async_copy(x2_hbm.at[gv.at[gt + b]],
                                      rows.at[b], gsem[b]).wait()
                pltpu.async_copy(rows.at[b], agg.at[dv.at[gt + b]],
                                 ssem[b], add=True)
            for b in range(NBUF):
                pltpu.make_async_copy(rows.at[b], agg.at[dv.at[gt + b]],
                                      ssem[b]).wait()
        plsc.subcore_barrier()
        pltpu.sync_copy(agg.at[pl.ds(r0, ROWS_PT)],
                        out_hbm.at[c, pl.ds(r0, ROWS_PT)])

        @pl.when(s == 0)
        def _():
            pltpu.sync_copy(agg.at[pl.ds(ROWS_PT * NS, TAIL)],
                            out_hbm.at[c, pl.ds(ROWS_PT * NS, TAIL)])

    return k(x2, xh, gidx, didx)


BLK = 1000


def _tc_body(a_ref, wg_ref, bg_ref, w0_ref, b0_ref, w1_ref, b1_ref,
             emb_ref, dlt_ref):
    dn = (((1,), (0,)), ((), ()))
    h0 = a_ref[0]
    h1 = a_ref[1]
    acc = lax.dot_general(h0, wg_ref[:HALF, :], dn,
                          preferred_element_type=jnp.float32,
                          precision=lax.Precision.HIGHEST)
    acc = acc + lax.dot_general(h1, wg_ref[HALF:, :], dn,
                                preferred_element_type=jnp.float32,
                                precision=lax.Precision.HIGHEST)
    e = jnp.maximum(acc + bg_ref[...], 0.0)
    emb_ref[...] = e
    t = lax.dot_general(e, w0_ref[...], dn,
                        preferred_element_type=jnp.float32,
                        precision=lax.Precision.HIGHEST) + b0_ref[...]
    t = jnp.where(t >= 0.0, t, 0.01 * t)
    dlt_ref[...] = lax.dot_general(t, w1_ref[...], dn,
                                   preferred_element_type=jnp.float32,
                                   precision=lax.Precision.HIGHEST) + b1_ref[...]


def _tc_dense(agg2, W_gcn, b_gcn, W0, b0, W1, b1):
    return pl.pallas_call(
        _tc_body,
        grid=(N // BLK,),
        in_specs=[
            pl.BlockSpec((NC, BLK, HALF), lambda i: (0, i, 0)),
            pl.BlockSpec((D, PC), lambda i: (0, 0)),
            pl.BlockSpec((1, PC), lambda i: (0, 0)),
            pl.BlockSpec((PC, NH), lambda i: (0, 0)),
            pl.BlockSpec((1, NH), lambda i: (0, 0)),
            pl.BlockSpec((NH, PC), lambda i: (0, 0)),
            pl.BlockSpec((1, PC), lambda i: (0, 0)),
        ],
        out_specs=(
            pl.BlockSpec((BLK, PC), lambda i: (i, 0)),
            pl.BlockSpec((BLK, PC), lambda i: (i, 0)),
        ),
        out_shape=(
            jax.ShapeDtypeStruct((N, PC), jnp.float32),
            jax.ShapeDtypeStruct((N, PC), jnp.float32),
        ),
    )(agg2, W_gcn, b_gcn.reshape(1, PC), W0, b0.reshape(1, NH),
      W1, b1.reshape(1, PC))


def kernel(x, edge_index, W_gcn, b_gcn, W0, b0, W1, b1):
    ei = edge_index.astype(jnp.int32)
    src, dst = ei[0], ei[1]
    pad = EPAD - E
    gidx = jnp.stack([2 * src, 2 * src + 1])
    gidx = jnp.pad(gidx, ((0, 0), (0, pad)))
    gidx = gidx.reshape(NC, NS * NCHUNK, K)
    didx = jnp.pad(dst, (0, pad), constant_values=N)
    didx = didx.reshape(NS * NCHUNK, K)
    x2 = x.reshape(2 * N, HALF)
    xh = x.reshape(N, 2, HALF).transpose(1, 0, 2)
    agg2 = _sc_segment_sum(x2, xh, gidx, didx)
    x_embedding, x_delta_hat = _tc_dense(agg2, W_gcn, b_gcn, W0, b0, W1, b1)
    return (x_embedding, x_delta_hat)


# speedup vs baseline: 1.1312x; 1.1312x over previous
import functools

import jax
import jax.numpy as jnp
from jax import lax
from jax.experimental import pallas as pl
from jax.experimental.pallas import tpu as pltpu
from jax.experimental.pallas import tpu_sc as plsc

N, E, D, PC, NH = 10000, 160000, 256, 256, 512
HALF = 128
NC, NS = 2, 16
K = 128
NBUF = 2
NPASS = 2
EPT = -(-E // (NS * K * NBUF * NPASS)) * K * NBUF * NPASS
EPAD = EPT * NS
NCHUNK = EPT // K
PCH = NCHUNK // NPASS
ROWS_PT = (N // NS) // 8 * 8
TAIL = N - ROWS_PT * NS
APAD = 8


def _sc_segment_sum(x2, xh, gidx, didx):
    mesh = plsc.VectorSubcoreMesh(core_axis_name="c", subcore_axis_name="s")

    @functools.partial(
        pl.kernel,
        mesh=mesh,
        out_type=jax.ShapeDtypeStruct((NC, N, HALF), jnp.float32),
        scratch_types=[
            pltpu.VMEM((PCH, K), jnp.int32),
            pltpu.VMEM((PCH, K), jnp.int32),
            pltpu.VMEM((NBUF, K, HALF), jnp.float32),
            pltpu.VMEM_SHARED((N + APAD, HALF), jnp.float32),
        ] + [pltpu.SemaphoreType.DMA] * (2 * NBUF),
    )
    def k(x2_hbm, xh_hbm, gidx_hbm, didx_hbm, out_hbm,
          gv, dv, rows, agg, *sems):
        gsem, ssem = sems[:NBUF], sems[NBUF:]
        c = lax.axis_index("c")
        s = lax.axis_index("s")
        r0 = s * ROWS_PT
        pltpu.sync_copy(xh_hbm.at[c, pl.ds(r0, ROWS_PT)],
                        agg.at[pl.ds(r0, ROWS_PT)])

        @pl.when(s == 0)
        def _():
            pltpu.sync_copy(xh_hbm.at[c, pl.ds(ROWS_PT * NS, TAIL)],
                            agg.at[pl.ds(ROWS_PT * NS, TAIL)])

        plsc.subcore_barrier()

        for p in range(NPASS):
            cb = s * NCHUNK + p * PCH
            pltpu.sync_copy(gidx_hbm.at[c, pl.ds(cb, PCH)], gv)
            pltpu.sync_copy(didx_hbm.at[pl.ds(cb, PCH)], dv)

            pltpu.async_copy(x2_hbm.at[gv.at[0]], rows.at[0], gsem[0])

            def outer(i, carry):
                for b in range(NBUF):
                    j = i * NBUF + b
                    pltpu.make_async_copy(x2_hbm.at[gv.at[j]],
                                          rows.at[b], gsem[b]).wait()
                    pltpu.async_copy(x2_hbm.at[gv.at[j + 1]],
                                     rows.at[1 - b], gsem[1 - b])
                    pltpu.sync_copy(rows.at[b], agg.at[dv.at[j]], add=True)
                return carry

            lax.fori_loop(0, PCH // NBUF - 1, outer, 0)
            gt = PCH - NBUF
            pltpu.make_async_copy(x2_hbm.at[gv.at[gt]],
                                  rows.at[0], gsem[0]).wait()
            pltpu.async_copy(x2_hbm.at[gv.at[gt + 1]], rows.at[1], gsem[1])
            pltpu.sync_copy(rows.at[0], agg.at[dv.at[gt]], add=True)
            pltpu.make_async_copy(x2_hbm.at[gv.at[gt + 1]],
                                  rows.at[1], gsem[1]).wait()
            pltpu.sync_copy(rows.at[1], agg.at[dv.at[gt + 1]], add=True)
        plsc.subcore_barrier()
        pltpu.sync_copy(agg.at[pl.ds(r0, ROWS_PT)],
                        out_hbm.at[c, pl.ds(r0, ROWS_PT)])

        @pl.when(s == 0)
        def _():
            pltpu.sync_copy(agg.at[pl.ds(ROWS_PT * NS, TAIL)],
                            out_hbm.at[c, pl.ds(ROWS_PT * NS, TAIL)])

    return k(x2, xh, gidx, didx)


BLK = 1000


def _tc_body(a_ref, wg_ref, bg_ref, w0_ref, b0_ref, w1_ref, b1_ref,
             emb_ref, dlt_ref):
    dn = (((1,), (0,)), ((), ()))
    h0 = a_ref[0]
    h1 = a_ref[1]
    acc = lax.dot_general(h0, wg_ref[:HALF, :], dn,
                          preferred_element_type=jnp.float32)
    acc = acc + lax.dot_general(h1, wg_ref[HALF:, :], dn,
                                preferred_element_type=jnp.float32)
    e = jnp.maximum(acc + bg_ref[...], 0.0)
    emb_ref[...] = e
    t = lax.dot_general(e, w0_ref[...], dn,
                        preferred_element_type=jnp.float32) + b0_ref[...]
    t = jnp.where(t >= 0.0, t, 0.01 * t)
    dlt_ref[...] = lax.dot_general(t, w1_ref[...], dn,
                                   preferred_element_type=jnp.float32) + b1_ref[...]


def _tc_dense(agg2, W_gcn, b_gcn, W0, b0, W1, b1):
    return pl.pallas_call(
        _tc_body,
        grid=(N // BLK,),
        in_specs=[
            pl.BlockSpec((NC, BLK, HALF), lambda i: (0, i, 0)),
            pl.BlockSpec((D, PC), lambda i: (0, 0)),
            pl.BlockSpec((1, PC), lambda i: (0, 0)),
            pl.BlockSpec((PC, NH), lambda i: (0, 0)),
            pl.BlockSpec((1, NH), lambda i: (0, 0)),
            pl.BlockSpec((NH, PC), lambda i: (0, 0)),
            pl.BlockSpec((1, PC), lambda i: (0, 0)),
        ],
        out_specs=(
            pl.BlockSpec((BLK, PC), lambda i: (i, 0)),
            pl.BlockSpec((BLK, PC), lambda i: (i, 0)),
        ),
        out_shape=(
            jax.ShapeDtypeStruct((N, PC), jnp.float32),
            jax.ShapeDtypeStruct((N, PC), jnp.float32),
        ),
    )(agg2, W_gcn, b_gcn.reshape(1, PC), W0, b0.reshape(1, NH),
      W1, b1.reshape(1, PC))


def kernel(x, edge_index, W_gcn, b_gcn, W0, b0, W1, b1):
    ei = edge_index.astype(jnp.int32)
    src, dst = ei[0], ei[1]
    pad = EPAD - E
    gidx = jnp.stack([2 * src, 2 * src + 1])
    gidx = jnp.pad(gidx, ((0, 0), (0, pad)))
    gidx = gidx.reshape(NC, NS * NCHUNK, K)
    didx = jnp.pad(dst, (0, pad), constant_values=N)
    didx = didx.reshape(NS * NCHUNK, K)
    x2 = x.reshape(2 * N, HALF)
    xh = x.reshape(N, 2, HALF).transpose(1, 0, 2)
    agg2 = _sc_segment_sum(x2, xh, gidx, didx)
    x_embedding, x_delta_hat = _tc_dense(agg2, W_gcn, b_gcn, W0, b0, W1, b1)
    return (x_embedding, x_delta_hat)


# speedup vs baseline: 1.1470x; 1.0140x over previous
import functools

import jax
import jax.numpy as jnp
from jax import lax
from jax.experimental import pallas as pl
from jax.experimental.pallas import tpu as pltpu
from jax.experimental.pallas import tpu_sc as plsc

N, E, D, PC, NH = 10000, 160000, 256, 256, 512
HALF = 128
NC, NS = 2, 16
K = 128
NBUF = 2
NPASS = 2
EPT = -(-E // (NS * K * NBUF * NPASS)) * K * NBUF * NPASS
EPAD = EPT * NS
NCHUNK = EPT // K
PCH = NCHUNK // NPASS
ROWS_PT = (N // NS) // 8 * 8
TAIL = N - ROWS_PT * NS
APAD = 8


def _sc_segment_sum(x2, x, gidx, didx):
    mesh = plsc.VectorSubcoreMesh(core_axis_name="c", subcore_axis_name="s")

    @functools.partial(
        pl.kernel,
        mesh=mesh,
        out_type=jax.ShapeDtypeStruct((NC, N, HALF), jnp.float32),
        scratch_types=[
            pltpu.VMEM((PCH, K), jnp.int32),
            pltpu.VMEM((PCH, K), jnp.int32),
            pltpu.VMEM((NBUF, K, HALF), jnp.float32),
            pltpu.VMEM_SHARED((N + APAD, HALF), jnp.float32),
        ] + [pltpu.SemaphoreType.DMA] * (2 * NBUF),
    )
    def k(x2_hbm, x_hbm, gidx_hbm, didx_hbm, out_hbm,
          gv, dv, rows, agg, *sems):
        gsem, ssem = sems[:NBUF], sems[NBUF:]
        c = lax.axis_index("c")
        s = lax.axis_index("s")
        r0 = s * ROWS_PT
        pltpu.sync_copy(x_hbm.at[pl.ds(r0, ROWS_PT), pl.ds(c * HALF, HALF)],
                        agg.at[pl.ds(r0, ROWS_PT)])

        @pl.when(s == 0)
        def _():
            pltpu.sync_copy(
                x_hbm.at[pl.ds(ROWS_PT * NS, TAIL), pl.ds(c * HALF, HALF)],
                agg.at[pl.ds(ROWS_PT * NS, TAIL)])

        plsc.subcore_barrier()

        def gather(j, b):
            pltpu.async_copy(x2_hbm.at[gv.at[j]], rows.at[b], gsem[b])

        def gather_wait(b):
            pltpu.make_async_copy(x2_hbm.at[gv.at[0]], rows.at[b],
                                  gsem[b]).wait()

        def scatter(j, b):
            pltpu.async_copy(rows.at[b], agg.at[dv.at[j]], ssem[b], add=True)

        def scatter_wait(b):
            pltpu.make_async_copy(rows.at[b], agg.at[dv.at[0]],
                                  ssem[b]).wait()

        for p in range(NPASS):
            cb = s * NCHUNK + p * PCH
            pltpu.sync_copy(gidx_hbm.at[c, pl.ds(cb, PCH)], gv)
            pltpu.sync_copy(didx_hbm.at[pl.ds(cb, PCH)], dv)

            gather(0, 0)
            gather_wait(0)
            scatter(0, 0)
            gather(1, 1)
            gather_wait(1)
            scatter(1, 1)
            scatter_wait(0)
            gather(2, 0)

            def step(j, b):
                gather_wait(b)
                scatter(j, b)
                scatter_wait(1 - b)
                gather(j + 1, 1 - b)

            def outer(i, carry):
                step(2 * i, 0)
                step(2 * i + 1, 1)
                return carry

            lax.fori_loop(1, PCH // 2 - 1, outer, 0)
            gather_wait(0)
            scatter(PCH - 2, 0)
            scatter_wait(1)
            gather(PCH - 1, 1)
            gather_wait(1)
            scatter(PCH - 1, 1)
            scatter_wait(0)
            scatter_wait(1)
        plsc.subcore_barrier()
        pltpu.sync_copy(agg.at[pl.ds(r0, ROWS_PT)],
                        out_hbm.at[c, pl.ds(r0, ROWS_PT)])

        @pl.when(s == 0)
        def _():
            pltpu.sync_copy(agg.at[pl.ds(ROWS_PT * NS, TAIL)],
                            out_hbm.at[c, pl.ds(ROWS_PT * NS, TAIL)])

    return k(x2, x, gidx, didx)


BLK = 1000


def _tc_body(a_ref, wg_ref, bg_ref, w0_ref, b0_ref, w1_ref, b1_ref,
             emb_ref, dlt_ref):
    dn = (((1,), (0,)), ((), ()))
    h0 = a_ref[0]
    h1 = a_ref[1]
    acc = lax.dot_general(h0, wg_ref[:HALF, :], dn,
                          preferred_element_type=jnp.float32)
    acc = acc + lax.dot_general(h1, wg_ref[HALF:, :], dn,
                                preferred_element_type=jnp.float32)
    e = jnp.maximum(acc + bg_ref[...], 0.0)
    emb_ref[...] = e
    t = lax.dot_general(e, w0_ref[...], dn,
                        preferred_element_type=jnp.float32) + b0_ref[...]
    t = jnp.where(t >= 0.0, t, 0.01 * t)
    dlt_ref[...] = lax.dot_general(t, w1_ref[...], dn,
                                   preferred_element_type=jnp.float32) + b1_ref[...]


def _tc_dense(agg2, W_gcn, b_gcn, W0, b0, W1, b1):
    return pl.pallas_call(
        _tc_body,
        grid=(N // BLK,),
        in_specs=[
            pl.BlockSpec((NC, BLK, HALF), lambda i: (0, i, 0)),
            pl.BlockSpec((D, PC), lambda i: (0, 0)),
            pl.BlockSpec((1, PC), lambda i: (0, 0)),
            pl.BlockSpec((PC, NH), lambda i: (0, 0)),
            pl.BlockSpec((1, NH), lambda i: (0, 0)),
            pl.BlockSpec((NH, PC), lambda i: (0, 0)),
            pl.BlockSpec((1, PC), lambda i: (0, 0)),
        ],
        out_specs=(
            pl.BlockSpec((BLK, PC), lambda i: (i, 0)),
            pl.BlockSpec((BLK, PC), lambda i: (i, 0)),
        ),
        out_shape=(
            jax.ShapeDtypeStruct((N, PC), jnp.float32),
            jax.ShapeDtypeStruct((N, PC), jnp.float32),
        ),
    )(agg2, W_gcn, b_gcn.reshape(1, PC), W0, b0.reshape(1, NH),
      W1, b1.reshape(1, PC))


def kernel(x, edge_index, W_gcn, b_gcn, W0, b0, W1, b1):
    ei = edge_index.astype(jnp.int32)
    src, dst = ei[0], ei[1]
    pad = EPAD - E
    gidx = jnp.stack([2 * src, 2 * src + 1])
    gidx = jnp.pad(gidx, ((0, 0), (0, pad)))
    gidx = gidx.reshape(NC, NS * NCHUNK, K)
    didx = jnp.pad(dst, (0, pad), constant_values=N)
    didx = didx.reshape(NS * NCHUNK, K)
    x2 = x.reshape(2 * N, HALF)
    agg2 = _sc_segment_sum(x2, x, gidx, didx)
    x_embedding, x_delta_hat = _tc_dense(agg2, W_gcn, b_gcn, W0, b0, W1, b1)
    return (x_embedding, x_delta_hat)


# speedup vs baseline: 1.1718x; 1.0216x over previous
import functools

import jax
import jax.numpy as jnp
from jax import lax
from jax.experimental import pallas as pl
from jax.experimental.pallas import tpu as pltpu
from jax.experimental.pallas import tpu_sc as plsc

N, E, D, PC, NH = 10000, 160000, 256, 256, 512
HALF = 128
NC, NS = 2, 16
K = 64
NBUF = 4
NPASS = 4
EPT = -(-E // (NS * K * NBUF * NPASS)) * K * NBUF * NPASS
EPAD = EPT * NS
NCHUNK = EPT // K
PCH = NCHUNK // NPASS
ROWS_PT = (N // NS) // 8 * 8
TAIL = N - ROWS_PT * NS
APAD = 8


def _sc_segment_sum(x2, x, gidx, didx):
    mesh = plsc.VectorSubcoreMesh(core_axis_name="c", subcore_axis_name="s")

    @functools.partial(
        pl.kernel,
        mesh=mesh,
        out_type=jax.ShapeDtypeStruct((NC, N, HALF), jnp.float32),
        scratch_types=[
            pltpu.VMEM((PCH, K), jnp.int32),
            pltpu.VMEM((PCH, K), jnp.int32),
            pltpu.VMEM((NBUF, K, HALF), jnp.float32),
            pltpu.VMEM_SHARED((N + APAD, HALF), jnp.float32),
        ] + [pltpu.SemaphoreType.DMA] * (2 * NBUF),
    )
    def k(x2_hbm, x_hbm, gidx_hbm, didx_hbm, out_hbm,
          gv, dv, rows, agg, *sems):
        gsem, ssem = sems[:NBUF], sems[NBUF:]
        c = lax.axis_index("c")
        s = lax.axis_index("s")
        r0 = s * ROWS_PT
        pltpu.sync_copy(x_hbm.at[pl.ds(r0, ROWS_PT), pl.ds(c * HALF, HALF)],
                        agg.at[pl.ds(r0, ROWS_PT)])

        @pl.when(s == 0)
        def _():
            pltpu.sync_copy(
                x_hbm.at[pl.ds(ROWS_PT * NS, TAIL), pl.ds(c * HALF, HALF)],
                agg.at[pl.ds(ROWS_PT * NS, TAIL)])

        plsc.subcore_barrier()

        def gather(j, b):
            pltpu.async_copy(x2_hbm.at[gv.at[j]], rows.at[b], gsem[b])

        def gather_wait(b):
            pltpu.make_async_copy(x2_hbm.at[gv.at[0]], rows.at[b],
                                  gsem[b]).wait()

        def scatter(j, b):
            pltpu.async_copy(rows.at[b], agg.at[dv.at[j]], ssem[b], add=True)

        def scatter_wait(b):
            pltpu.make_async_copy(rows.at[b], agg.at[dv.at[0]],
                                  ssem[b]).wait()

        G = NBUF - 1

        for p in range(NPASS):
            cb = s * NCHUNK + p * PCH
            pltpu.sync_copy(gidx_hbm.at[c, pl.ds(cb, PCH)], gv)
            pltpu.sync_copy(didx_hbm.at[pl.ds(cb, PCH)], dv)

            for b in range(G):
                gather(b, b)
            gather_wait(0)
            scatter(0, 0)
            gather(G, G)
            for b in range(1, NBUF):
                gather_wait(b)
                scatter(b, b)
                scatter_wait(b - 1)
                gather(b + G, b - 1)

            def outer(i, carry):
                for b in range(NBUF):
                    j = i * NBUF + b
                    t = (b + G) % NBUF
                    gather_wait(b)
                    scatter(j, b)
                    scatter_wait(t)
                    gather(j + G, t)
                return carry

            lax.fori_loop(1, PCH // NBUF - 1, outer, 0)
            jl = PCH - NBUF
            gather_wait(0)
            scatter(jl, 0)
            scatter_wait(G % NBUF)
            gather(jl + G, G % NBUF)
            for b in range(1, NBUF):
                gather_wait(b)
                scatter(jl + b, b)
                scatter_wait(b - 1)
            scatter_wait(NBUF - 1)
        plsc.subcore_barrier()
        pltpu.sync_copy(agg.at[pl.ds(r0, ROWS_PT)],
                        out_hbm.at[c, pl.ds(r0, ROWS_PT)])

        @pl.when(s == 0)
        def _():
            pltpu.sync_copy(agg.at[pl.ds(ROWS_PT * NS, TAIL)],
                            out_hbm.at[c, pl.ds(ROWS_PT * NS, TAIL)])

    return k(x2, x, gidx, didx)


BLK = 1000


def _tc_body(a_ref, wg_ref, bg_ref, w0_ref, b0_ref, w1_ref, b1_ref,
             emb_ref, dlt_ref):
    dn = (((1,), (0,)), ((), ()))
    h0 = a_ref[0]
    h1 = a_ref[1]
    acc = lax.dot_general(h0, wg_ref[:HALF, :], dn,
                          preferred_element_type=jnp.float32)
    acc = acc + lax.dot_general(h1, wg_ref[HALF:, :], dn,
                                preferred_element_type=jnp.float32)
    e = jnp.maximum(acc + bg_ref[...], 0.0)
    emb_ref[...] = e
    t = lax.dot_general(e, w0_ref[...], dn,
                        preferred_element_type=jnp.float32) + b0_ref[...]
    t = jnp.where(t >= 0.0, t, 0.01 * t)
    dlt_ref[...] = lax.dot_general(t, w1_ref[...], dn,
                                   preferred_element_type=jnp.float32) + b1_ref[...]


def _tc_dense(agg2, W_gcn, b_gcn, W0, b0, W1, b1):
    return pl.pallas_call(
        _tc_body,
        grid=(N // BLK,),
        in_specs=[
            pl.BlockSpec((NC, BLK, HALF), lambda i: (0, i, 0)),
            pl.BlockSpec((D, PC), lambda i: (0, 0)),
            pl.BlockSpec((1, PC), lambda i: (0, 0)),
            pl.BlockSpec((PC, NH), lambda i: (0, 0)),
            pl.BlockSpec((1, NH), lambda i: (0, 0)),
            pl.BlockSpec((NH, PC), lambda i: (0, 0)),
            pl.BlockSpec((1, PC), lambda i: (0, 0)),
        ],
        out_specs=(
            pl.BlockSpec((BLK, PC), lambda i: (i, 0)),
            pl.BlockSpec((BLK, PC), lambda i: (i, 0)),
        ),
        out_shape=(
            jax.ShapeDtypeStruct((N, PC), jnp.float32),
            jax.ShapeDtypeStruct((N, PC), jnp.float32),
        ),
    )(agg2, W_gcn, b_gcn.reshape(1, PC), W0, b0.reshape(1, NH),
      W1, b1.reshape(1, PC))


def kernel(x, edge_index, W_gcn, b_gcn, W0, b0, W1, b1):
    ei = edge_index.astype(jnp.int32)
    src, dst = ei[0], ei[1]
    pad = EPAD - E
    gidx = jnp.stack([2 * src, 2 * src + 1])
    gidx = jnp.pad(gidx, ((0, 0), (0, pad)))
    gidx = gidx.reshape(NC, NS * NCHUNK, K)
    didx = jnp.pad(dst, (0, pad), constant_values=N)
    didx = didx.reshape(NS * NCHUNK, K)
    x2 = x.reshape(2 * N, HALF)
    agg2 = _sc_segment_sum(x2, x, gidx, didx)
    x_embedding, x_delta_hat = _tc_dense(agg2, W_gcn, b_gcn, W0, b0, W1, b1)
    return (x_embedding, x_delta_hat)


# speedup vs baseline: 1.1760x; 1.0036x over previous
import functools

import jax
import jax.numpy as jnp
from jax import lax
from jax.experimental import pallas as pl
from jax.experimental.pallas import tpu as pltpu
from jax.experimental.pallas import tpu_sc as plsc

N, E, D, PC, NH = 10000, 160000, 256, 256, 512
HALF = 128
NC, NS = 2, 16
K = 64
NBUF = 4
NPASS = 4
EPT = -(-E // (NS * K * NBUF * NPASS)) * K * NBUF * NPASS
EPAD = EPT * NS
NCHUNK = EPT // K
PCH = NCHUNK // NPASS
ROWS_PT = (N // NS) // 8 * 8
TAIL = N - ROWS_PT * NS
APAD = 8


def _sc_segment_sum(x2, x, gidx, didx):
    mesh = plsc.VectorSubcoreMesh(core_axis_name="c", subcore_axis_name="s")

    @functools.partial(
        pl.kernel,
        mesh=mesh,
        out_type=jax.ShapeDtypeStruct((NC, N, HALF), jnp.float32),
        scratch_types=[
            pltpu.VMEM((PCH, K), jnp.int32),
            pltpu.VMEM((PCH, K), jnp.int32),
            pltpu.VMEM((NBUF, K, HALF), jnp.float32),
            pltpu.VMEM_SHARED((N + APAD, HALF), jnp.float32),
        ] + [pltpu.SemaphoreType.DMA] * (2 * NBUF),
    )
    def k(x2_hbm, x_hbm, gidx_hbm, didx_hbm, out_hbm,
          gv, dv, rows, agg, *sems):
        gsem, ssem = sems[:NBUF], sems[NBUF:]
        c = lax.axis_index("c")
        s = lax.axis_index("s")
        r0 = s * ROWS_PT
        pltpu.sync_copy(x_hbm.at[pl.ds(r0, ROWS_PT), pl.ds(c * HALF, HALF)],
                        agg.at[pl.ds(r0, ROWS_PT)])

        @pl.when(s == 0)
        def _():
            pltpu.sync_copy(
                x_hbm.at[pl.ds(ROWS_PT * NS, TAIL), pl.ds(c * HALF, HALF)],
                agg.at[pl.ds(ROWS_PT * NS, TAIL)])

        plsc.subcore_barrier()

        def gather(j, b):
            pltpu.async_copy(x2_hbm.at[gv.at[j]], rows.at[b], gsem[b])

        def gather_wait(b):
            pltpu.make_async_copy(x2_hbm.at[gv.at[0]], rows.at[b],
                                  gsem[b]).wait()

        def scatter(j, b):
            pltpu.async_copy(rows.at[b], agg.at[dv.at[j]], ssem[b], add=True)

        def scatter_wait(b):
            pltpu.make_async_copy(rows.at[b], agg.at[dv.at[0]],
                                  ssem[b]).wait()

        G = NBUF - 1

        for p in range(NPASS):
            cb = s * NCHUNK + p * PCH
            pltpu.sync_copy(gidx_hbm.at[c, pl.ds(cb, PCH)], gv)
            pltpu.sync_copy(didx_hbm.at[pl.ds(cb, PCH)], dv)

            for b in range(G):
                gather(b, b)
            gather_wait(0)
            scatter(0, 0)
            gather(G, G)
            for b in range(1, NBUF):
                gather_wait(b)
                scatter(b, b)
                scatter_wait(b - 1)
                gather(b + G, b - 1)

            def outer(i, carry):
                for b in range(NBUF):
                    j = i * NBUF + b
                    t = (b + G) % NBUF
                    gather_wait(b)
                    scatter(j, b)
                    scatter_wait(t)
                    gather(j + G, t)
                return carry

            lax.fori_loop(1, PCH // NBUF - 1, outer, 0)
            jl = PCH - NBUF
            gather_wait(0)
            scatter(jl, 0)
            scatter_wait(G % NBUF)
            gather(jl + G, G % NBUF)
            for b in range(1, NBUF):
                gather_wait(b)
                scatter(jl + b, b)
                scatter_wait(b - 1)
            scatter_wait(NBUF - 1)
        plsc.subcore_barrier()
        pltpu.sync_copy(agg.at[pl.ds(r0, ROWS_PT)],
                        out_hbm.at[c, pl.ds(r0, ROWS_PT)])

        @pl.when(s == 0)
        def _():
            pltpu.sync_copy(agg.at[pl.ds(ROWS_PT * NS, TAIL)],
                            out_hbm.at[c, pl.ds(ROWS_PT * NS, TAIL)])

    return k(x2, x, gidx, didx)


BLK = 2000


def _tc_body(a_ref, wg_ref, bg_ref, w0_ref, b0_ref, w1_ref, b1_ref,
             emb_ref, dlt_ref):
    dn = (((1,), (0,)), ((), ()))
    h0 = a_ref[0]
    h1 = a_ref[1]
    acc = lax.dot_general(h0, wg_ref[:HALF, :], dn,
                          preferred_element_type=jnp.float32)
    acc = acc + lax.dot_general(h1, wg_ref[HALF:, :], dn,
                                preferred_element_type=jnp.float32)
    e = jnp.maximum(acc + bg_ref[...], 0.0)
    emb_ref[...] = e
    t = lax.dot_general(e, w0_ref[...], dn,
                        preferred_element_type=jnp.float32) + b0_ref[...]
    t = jnp.where(t >= 0.0, t, 0.01 * t)
    dlt_ref[...] = lax.dot_general(t, w1_ref[...], dn,
                                   preferred_element_type=jnp.float32) + b1_ref[...]


def _tc_dense(agg2, W_gcn, b_gcn, W0, b0, W1, b1):
    return pl.pallas_call(
        _tc_body,
        grid=(N // BLK,),
        in_specs=[
            pl.BlockSpec((NC, BLK, HALF), lambda i: (0, i, 0)),
            pl.BlockSpec((D, PC), lambda i: (0, 0)),
            pl.BlockSpec((1, PC), lambda i: (0, 0)),
            pl.BlockSpec((PC, NH), lambda i: (0, 0)),
            pl.BlockSpec((1, NH), lambda i: (0, 0)),
            pl.BlockSpec((NH, PC), lambda i: (0, 0)),
            pl.BlockSpec((1, PC), lambda i: (0, 0)),
        ],
        out_specs=(
            pl.BlockSpec((BLK, PC), lambda i: (i, 0)),
            pl.BlockSpec((BLK, PC), lambda i: (i, 0)),
        ),
        out_shape=(
            jax.ShapeDtypeStruct((N, PC), jnp.float32),
            jax.ShapeDtypeStruct((N, PC), jnp.float32),
        ),
    )(agg2, W_gcn, b_gcn.reshape(1, PC), W0, b0.reshape(1, NH),
      W1, b1.reshape(1, PC))


def kernel(x, edge_index, W_gcn, b_gcn, W0, b0, W1, b1):
    ei = edge_index.astype(jnp.int32)
    src, dst = ei[0], ei[1]
    pad = EPAD - E
    gidx = jnp.stack([2 * src, 2 * src + 1])
    gidx = jnp.pad(gidx, ((0, 0), (0, pad)))
    gidx = gidx.reshape(NC, NS * NCHUNK, K)
    didx = jnp.pad(dst, (0, pad), constant_values=N)
    didx = didx.reshape(NS * NCHUNK, K)
    x2 = x.reshape(2 * N, HALF)
    agg2 = _sc_segment_sum(x2, x, gidx, didx)
    x_embedding, x_delta_hat = _tc_dense(agg2, W_gcn, b_gcn, W0, b0, W1, b1)
    return (x_embedding, x_delta_hat)
